# baseline (device time: 7411 ns/iter reference)
import functools

import jax
import jax.numpy as jnp
from jax import lax
from jax.experimental import pallas as pl
from jax.experimental.pallas import tpu as pltpu

N_CHUNKS = 6


def kernel(x):
    m, n = x.shape
    rows = m // N_CHUNKS

    def body(x_hbm, out_ref, xv_ref, a_ref, send_ref, recv_ref,
             copy_sems, send_sem, recv_sem):
        my_x = lax.axis_index("x")
        my_y = lax.axis_index("y")
        peer = (my_x, 1 - my_y)

        barrier_sem = pltpu.get_barrier_semaphore()
        pl.semaphore_signal(
            barrier_sem, inc=1, device_id=peer,
            device_id_type=pl.DeviceIdType.MESH,
        )

        copies = []
        for c in range(N_CHUNKS):
            cp = pltpu.make_async_copy(
                x_hbm.at[pl.ds(c * rows, rows), :],
                xv_ref.at[pl.ds(c * rows, rows), :],
                copy_sems.at[c],
            )
            cp.start()
            copies.append(cp)

        for c in range(N_CHUNKS):
            copies[c].wait()
            xa = xv_ref[pl.ds(c * rows, rows), :]
            blocks = [xa[:, k * 128:(k + 1) * 128] for k in range(n // 128)]
            a_ref[pl.ds(c * rows, rows), :] = functools.reduce(
                jnp.maximum, blocks
            ).astype(jnp.bfloat16)

        send_ref[:, :] = jnp.max(a_ref[:, :], axis=1).reshape(send_ref.shape)

        pl.semaphore_wait(barrier_sem, 1)
        rdma = pltpu.make_async_remote_copy(
            src_ref=send_ref,
            dst_ref=recv_ref,
            send_sem=send_sem,
            recv_sem=recv_sem,
            device_id=peer,
            device_id_type=pl.DeviceIdType.MESH,
        )
        rdma.start()
        rdma.wait()

        out_ref[:, :] = jnp.maximum(
            send_ref[:, :], recv_ref[:, :]
        ).astype(jnp.float32)

    out = pl.pallas_call(
        body,
        out_shape=jax.ShapeDtypeStruct((m // 128, 128), jnp.float32),
        in_specs=[pl.BlockSpec(memory_space=pl.ANY)],
        out_specs=pl.BlockSpec(memory_space=pltpu.VMEM),
        scratch_shapes=[
            pltpu.VMEM((m, n), jnp.float32),
            pltpu.VMEM((m, 128), jnp.bfloat16),
            pltpu.VMEM((m // 128, 128), jnp.bfloat16),
            pltpu.VMEM((m // 128, 128), jnp.bfloat16),
            pltpu.SemaphoreType.DMA((N_CHUNKS,)),
            pltpu.SemaphoreType.DMA,
            pltpu.SemaphoreType.DMA,
        ],
        compiler_params=pltpu.CompilerParams(collective_id=0),
    )(x)
    return out.reshape(m, 1)


# device time: 6776 ns/iter; 1.0937x vs baseline; 1.0937x over previous
import jax
import jax.numpy as jnp
from jax import lax
from jax.experimental import pallas as pl
from jax.experimental.pallas import tpu as pltpu

N_CHUNKS = 6


def kernel(x):
    m, n = x.shape
    rows = m // N_CHUNKS
    prows = rows // 128

    def body(x_hbm, out_ref, xv_ref, recv_ref, copy_sems, send_sem, recv_sem):
        my_x = lax.axis_index("x")
        my_y = lax.axis_index("y")
        peer = (my_x, 1 - my_y)

        barrier_sem = pltpu.get_barrier_semaphore()
        pl.semaphore_signal(
            barrier_sem, inc=1, device_id=peer,
            device_id_type=pl.DeviceIdType.MESH,
        )

        copies = []
        for c in range(N_CHUNKS):
            cp = pltpu.make_async_copy(
                x_hbm.at[pl.ds(c * rows, rows), :],
                xv_ref.at[pl.ds(c * rows, rows), :],
                copy_sems.at[c],
            )
            cp.start()
            copies.append(cp)

        for c in range(N_CHUNKS):
            copies[c].wait()
            part = jnp.max(xv_ref[pl.ds(c * rows, rows), :], axis=1)
            out_ref[pl.ds(c * prows, prows), :] = part.reshape(prows, 128)

        pl.semaphore_wait(barrier_sem, 1)
        rdma = pltpu.make_async_remote_copy(
            src_ref=out_ref,
            dst_ref=recv_ref,
            send_sem=send_sem,
            recv_sem=recv_sem,
            device_id=peer,
            device_id_type=pl.DeviceIdType.MESH,
        )
        rdma.start()
        rdma.wait()

        out_ref[:, :] = jnp.maximum(out_ref[:, :], recv_ref[:, :])

    out = pl.pallas_call(
        body,
        out_shape=jax.ShapeDtypeStruct((m // 128, 128), jnp.float32),
        in_specs=[pl.BlockSpec(memory_space=pl.ANY)],
        out_specs=pl.BlockSpec(memory_space=pltpu.VMEM),
        scratch_shapes=[
            pltpu.VMEM((m, n), jnp.float32),
            pltpu.VMEM((m // 128, 128), jnp.float32),
            pltpu.SemaphoreType.DMA((N_CHUNKS,)),
            pltpu.SemaphoreType.DMA,
            pltpu.SemaphoreType.DMA,
        ],
        compiler_params=pltpu.CompilerParams(collective_id=0),
    )(pltpu.with_memory_space_constraint(x, pltpu.MemorySpace.HBM))
    return out.reshape(m, 1)


# device time: 6740 ns/iter; 1.0996x vs baseline; 1.0053x over previous
import jax
import jax.numpy as jnp
from jax import lax
from jax.experimental import pallas as pl
from jax.experimental.pallas import tpu as pltpu

N_CHUNKS = 6


def kernel(x):
    m, n = x.shape
    rows = m // N_CHUNKS
    prows = rows // 128

    def body(x_hbm, out_ref, xv_ref, recv_ref, copy_sems, send_sems, recv_sems):
        my_x = lax.axis_index("x")
        my_y = lax.axis_index("y")
        peer = (my_x, 1 - my_y)

        barrier_sem = pltpu.get_barrier_semaphore()
        pl.semaphore_signal(
            barrier_sem, inc=1, device_id=peer,
            device_id_type=pl.DeviceIdType.MESH,
        )

        copies = []
        for c in range(N_CHUNKS):
            cp = pltpu.make_async_copy(
                x_hbm.at[pl.ds(c * rows, rows), :],
                xv_ref.at[pl.ds(c * rows, rows), :],
                copy_sems.at[c],
            )
            cp.start()
            copies.append(cp)

        half = N_CHUNKS // 2
        rdmas = []
        for h in range(2):
            for c in range(h * half, (h + 1) * half):
                copies[c].wait()
                part = jnp.max(xv_ref[pl.ds(c * rows, rows), :], axis=1)
                out_ref[pl.ds(c * prows, prows), :] = part.reshape(prows, 128)
            if h == 0:
                pl.semaphore_wait(barrier_sem, 1)
            sl = pl.ds(h * half * prows, half * prows)
            rdma = pltpu.make_async_remote_copy(
                src_ref=out_ref.at[sl],
                dst_ref=recv_ref.at[sl],
                send_sem=send_sems.at[h],
                recv_sem=recv_sems.at[h],
                device_id=peer,
                device_id_type=pl.DeviceIdType.MESH,
            )
            rdma.start()
            rdmas.append(rdma)
        for rdma in rdmas:
            rdma.wait()

        out_ref[:, :] = jnp.maximum(out_ref[:, :], recv_ref[:, :])

    out = pl.pallas_call(
        body,
        out_shape=jax.ShapeDtypeStruct((m // 128, 128), jnp.float32),
        in_specs=[pl.BlockSpec(memory_space=pl.ANY)],
        out_specs=pl.BlockSpec(memory_space=pltpu.VMEM),
        scratch_shapes=[
            pltpu.VMEM((m, n), jnp.float32),
            pltpu.VMEM((m // 128, 128), jnp.float32),
            pltpu.SemaphoreType.DMA((N_CHUNKS,)),
            pltpu.SemaphoreType.DMA((2,)),
            pltpu.SemaphoreType.DMA((2,)),
        ],
        compiler_params=pltpu.CompilerParams(collective_id=0),
    )(pltpu.with_memory_space_constraint(x, pltpu.MemorySpace.HBM))
    return out.reshape(m, 1)
